# baseline (speedup 1.0000x reference)
import functools

import jax
import jax.numpy as jnp
from jax import lax
from jax.experimental import pallas as pl
from jax.experimental.pallas import tpu as pltpu
from jax.experimental.pallas import tpu_sc as plsc

TOKENS = 16384
DIM = 2048
N_EXPERTS = 16
K = 2
BT = 1024

NW = 32
TPW = TOKENS // NW
NG = TPW // 16


def _mm_block(x_ref, w_ref, logits_ref):
    logits_ref[...] = jax.lax.dot_general(
        x_ref[...], w_ref[...], (((1,), (1,)), ((), ())),
        preferred_element_type=jnp.float32,
    )


def _tc_logits(x, W):
    grid = (TOKENS // BT,)
    return pl.pallas_call(
        _mm_block,
        grid=grid,
        in_specs=[
            pl.BlockSpec((BT, DIM), lambda i: (i, 0)),
            pl.BlockSpec((N_EXPERTS, DIM), lambda i: (0, 0)),
        ],
        out_specs=pl.BlockSpec((BT, N_EXPERTS), lambda i: (i, 0)),
        out_shape=jax.ShapeDtypeStruct((TOKENS, N_EXPERTS), jnp.float32),
    )(x, W)


def _sc_route_body(logits_hbm, gates_hbm, vals_hbm, idx_hbm,
                   lbuf, gbuf, vbuf, ibuf):
    wid = lax.axis_index("s") * 2 + lax.axis_index("c")
    base = wid * TPW
    pltpu.sync_copy(logits_hbm.at[pl.ds(base * N_EXPERTS, TPW * N_EXPERTS)], lbuf)

    lane = lax.iota(jnp.int32, 16)

    def group(g, carry):
        row0 = g * 16
        rows = lane + row0
        lrows = rows * N_EXPERTS
        krows = rows * K
        cols = []
        for e in range(N_EXPERTS):
            cols.append(plsc.load_gather(lbuf, [lrows + e]))
        m = cols[0]
        for e in range(1, N_EXPERTS):
            m = jnp.maximum(m, cols[e])
        evecs = [jnp.exp(c - m) for c in cols]
        s = evecs[0]
        for e in range(1, N_EXPERTS):
            s = s + evecs[e]
        rinv = 1.0 / s
        v1 = jnp.full((16,), -jnp.inf, jnp.float32)
        v2 = jnp.full((16,), -jnp.inf, jnp.float32)
        i1 = jnp.zeros((16,), jnp.int32)
        i2 = jnp.zeros((16,), jnp.int32)
        for e in range(N_EXPERTS):
            ge = evecs[e] * rinv
            ce = jnp.full((16,), e, jnp.int32)
            plsc.store_scatter(gbuf, [lrows + e], ge)
            gt1 = ge > v1
            gt2 = ge > v2
            v2 = jnp.where(gt1, v1, jnp.where(gt2, ge, v2))
            i2 = jnp.where(gt1, i1, jnp.where(gt2, ce, i2))
            v1 = jnp.where(gt1, ge, v1)
            i1 = jnp.where(gt1, ce, i1)
        plsc.store_scatter(vbuf, [krows], v1)
        plsc.store_scatter(vbuf, [krows + 1], v2)
        plsc.store_scatter(ibuf, [krows], i1)
        plsc.store_scatter(ibuf, [krows + 1], i2)
        return carry

    lax.fori_loop(0, NG, group, 0)

    pltpu.sync_copy(gbuf, gates_hbm.at[pl.ds(base * N_EXPERTS, TPW * N_EXPERTS)])
    pltpu.sync_copy(vbuf, vals_hbm.at[pl.ds(base * K, TPW * K)])
    pltpu.sync_copy(ibuf, idx_hbm.at[pl.ds(base * K, TPW * K)])


@functools.partial(
    pl.kernel,
    out_type=[
        jax.ShapeDtypeStruct((TOKENS * N_EXPERTS,), jnp.float32),
        jax.ShapeDtypeStruct((TOKENS * K,), jnp.float32),
        jax.ShapeDtypeStruct((TOKENS * K,), jnp.int32),
    ],
    mesh=plsc.VectorSubcoreMesh(core_axis_name="c", subcore_axis_name="s"),
    compiler_params=pltpu.CompilerParams(needs_layout_passes=False),
    scratch_types=[
        pltpu.VMEM((TPW * N_EXPERTS,), jnp.float32),
        pltpu.VMEM((TPW * N_EXPERTS,), jnp.float32),
        pltpu.VMEM((TPW * K,), jnp.float32),
        pltpu.VMEM((TPW * K,), jnp.int32),
    ],
)
def _sc_route(logits_hbm, gates_hbm, vals_hbm, idx_hbm, lbuf, gbuf, vbuf, ibuf):
    _sc_route_body(logits_hbm, gates_hbm, vals_hbm, idx_hbm,
                   lbuf, gbuf, vbuf, ibuf)


@jax.jit
def kernel(x, W):
    logits = _tc_logits(x, W)
    gates, vals, idx = _sc_route(logits.reshape(-1))
    return (
        gates.reshape(TOKENS, N_EXPERTS),
        vals.reshape(TOKENS, K),
        idx.reshape(TOKENS, K),
    )


# speedup vs baseline: 2.7117x; 2.7117x over previous
import jax
import jax.numpy as jnp
from jax.experimental import pallas as pl
from jax.experimental.pallas import tpu as pltpu

TOKENS = 16384
DIM = 2048
N_EXPERTS = 16
K = 2
BT = 1024


def _router_block(x_ref, w_ref, gatesT_ref, valsT_ref, idxT_ref):
    logitsT = jax.lax.dot_general(
        w_ref[...], x_ref[...], (((1,), (1,)), ((), ())),
        preferred_element_type=jnp.float32,
    )
    m = jnp.max(logitsT, axis=0, keepdims=True)
    e = jnp.exp(logitsT - m)
    s = jnp.sum(e, axis=0, keepdims=True)
    gatesT = e / s
    gatesT_ref[...] = gatesT
    iota = jax.lax.broadcasted_iota(jnp.int32, gatesT.shape, 0)
    v1 = jnp.max(gatesT, axis=0, keepdims=True)
    i1 = jnp.min(jnp.where(gatesT == v1, iota, N_EXPERTS), axis=0, keepdims=True)
    masked = jnp.where(iota == i1, -jnp.inf, gatesT)
    v2 = jnp.max(masked, axis=0, keepdims=True)
    i2 = jnp.min(jnp.where(masked == v2, iota, N_EXPERTS), axis=0, keepdims=True)
    valsT_ref[...] = jnp.concatenate([v1, v2], axis=0)
    idxT_ref[...] = jnp.concatenate([i1, i2], axis=0)


@jax.jit
def kernel(x, W):
    grid = (TOKENS // BT,)
    gatesT, valsT, idxT = pl.pallas_call(
        _router_block,
        grid=grid,
        in_specs=[
            pl.BlockSpec((BT, DIM), lambda i: (i, 0)),
            pl.BlockSpec((N_EXPERTS, DIM), lambda i: (0, 0)),
        ],
        out_specs=[
            pl.BlockSpec((N_EXPERTS, BT), lambda i: (0, i)),
            pl.BlockSpec((K, BT), lambda i: (0, i)),
            pl.BlockSpec((K, BT), lambda i: (0, i)),
        ],
        out_shape=[
            jax.ShapeDtypeStruct((N_EXPERTS, TOKENS), jnp.float32),
            jax.ShapeDtypeStruct((K, TOKENS), jnp.float32),
            jax.ShapeDtypeStruct((K, TOKENS), jnp.int32),
        ],
    )(x, W)
    return (gatesT.T, valsT.T, idxT.T)
